# baseline (speedup 1.0000x reference)
import functools

import jax
import jax.numpy as jnp
from jax import lax
from jax.experimental import pallas as pl
from jax.experimental.pallas import tpu as pltpu
from jax.experimental.pallas import tpu_sc as plsc


def _make_sc_gather(B, n_fields, vec, n_chunk):
    info = plsc.get_sparse_core_info()
    nc, ns = info.num_cores, info.num_subcores
    nw = nc * ns
    b_per_w = B // nw
    n_chunks = b_per_w // n_chunk
    mesh = plsc.VectorSubcoreMesh(core_axis_name="c", subcore_axis_name="s")

    @functools.partial(
        pl.kernel,
        mesh=mesh,
        compiler_params=pltpu.CompilerParams(use_tc_tiling_on_sc=False),
        out_type=[
            jax.ShapeDtypeStruct((B, vec), jnp.float32),
            jax.ShapeDtypeStruct((B, vec), jnp.float32),
            jax.ShapeDtypeStruct((B * n_fields, vec), jnp.float32),
        ],
        scratch_types=[
            pltpu.VMEM((n_chunk,), jnp.int32),
            pltpu.VMEM((n_chunk,), jnp.int32),
            pltpu.VMEM((n_chunk * n_fields,), jnp.int32),
            pltpu.VMEM((n_chunk, vec), jnp.float32),
            pltpu.VMEM((n_chunk, vec), jnp.float32),
            pltpu.VMEM((n_chunk * n_fields, vec), jnp.float32),
            pltpu.SemaphoreType.DMA,
            pltpu.SemaphoreType.DMA,
            pltpu.SemaphoreType.DMA,
        ],
    )
    def gather_kernel(idx_u_hbm, idx_i_hbm, idx_f_hbm,
                      user_hbm, item_hbm, feat_hbm,
                      out_u, out_i, out_f,
                      idx_u_v, idx_i_v, idx_f_v,
                      u_buf, i_buf, f_buf,
                      sem_u, sem_i, sem_f):
        wid = lax.axis_index("s") * nc + lax.axis_index("c")
        base = wid * b_per_w
        for c in range(n_chunks):
            r0 = base + c * n_chunk
            pltpu.sync_copy(idx_u_hbm.at[pl.ds(r0, n_chunk)], idx_u_v)
            pltpu.sync_copy(idx_i_hbm.at[pl.ds(r0, n_chunk)], idx_i_v)
            pltpu.sync_copy(
                idx_f_hbm.at[pl.ds(r0 * n_fields, n_chunk * n_fields)], idx_f_v)
            cu = pltpu.async_copy(user_hbm.at[idx_u_v], u_buf, sem_u)
            ci = pltpu.async_copy(item_hbm.at[idx_i_v], i_buf, sem_i)
            cf = pltpu.async_copy(feat_hbm.at[idx_f_v], f_buf, sem_f)
            cu.wait()
            ci.wait()
            cf.wait()
            pltpu.sync_copy(u_buf, out_u.at[pl.ds(r0, n_chunk)])
            pltpu.sync_copy(i_buf, out_i.at[pl.ds(r0, n_chunk)])
            pltpu.sync_copy(
                f_buf, out_f.at[pl.ds(r0 * n_fields, n_chunk * n_fields)])

    return gather_kernel


def _fm_body(u_ref, i_ref, f_ref, k_ref, w_ref, b_ref, o_ref):
    vec = u_ref.shape[1]
    kk = k_ref[...]
    wv = w_ref[...]
    s2 = jnp.sum(kk * kk, axis=1, keepdims=True)
    u = u_ref[...]
    it = i_ref[...]
    fe = f_ref[...]

    def mm(a, m):
        return jnp.dot(a, m, preferred_element_type=jnp.float32)

    p = (mm(u, kk[0:vec]) + mm(it, kk[vec:2 * vec]) + mm(fe, kk[2 * vec:]))
    lin = (mm(u, wv[0:vec]) + mm(it, wv[vec:2 * vec]) + mm(fe, wv[2 * vec:]))
    q = (mm(u * u, s2[0:vec]) + mm(it * it, s2[vec:2 * vec])
         + mm(fe * fe, s2[2 * vec:]))
    cross = 0.5 * (jnp.sum(p * p, axis=1, keepdims=True) - q)
    o_ref[...] = jax.nn.sigmoid(lin + b_ref[...] + cross)


def kernel(inputs, user_table, item_table, feat_tables, w, b, k_mat):
    B = inputs.shape[0]
    n_fields = feat_tables.shape[0]
    vocab = feat_tables.shape[1]
    vec = feat_tables.shape[2]
    total_dim = (2 + n_fields) * vec

    idx_u = inputs[:, 0].astype(jnp.int32)
    idx_i = inputs[:, 1].astype(jnp.int32)
    offs = (jnp.arange(n_fields, dtype=jnp.int32) * vocab)[None, :]
    idx_f = (inputs[:, 2:].astype(jnp.int32) + offs).reshape(-1)
    feat_flat = feat_tables.reshape(n_fields * vocab, vec)

    gather = _make_sc_gather(B, n_fields, vec, n_chunk=256)
    out_u, out_i, out_f = gather(idx_u, idx_i, idx_f,
                                 user_table, item_table, feat_flat)
    fe = out_f.reshape(B, n_fields * vec)

    bt = 1024
    b2 = jnp.reshape(b, (1, 1))
    y = pl.pallas_call(
        _fm_body,
        grid=(B // bt,),
        in_specs=[
            pl.BlockSpec((bt, vec), lambda i: (i, 0)),
            pl.BlockSpec((bt, vec), lambda i: (i, 0)),
            pl.BlockSpec((bt, n_fields * vec), lambda i: (i, 0)),
            pl.BlockSpec((total_dim, k_mat.shape[1]), lambda i: (0, 0)),
            pl.BlockSpec((total_dim, 1), lambda i: (0, 0)),
            pl.BlockSpec((1, 1), lambda i: (0, 0)),
        ],
        out_specs=pl.BlockSpec((bt, 1), lambda i: (i, 0)),
        out_shape=jax.ShapeDtypeStruct((B, 1), jnp.float32),
    )(out_u, out_i, fe, k_mat, w, b2)
    return y
